# baseline (speedup 1.0000x reference)
import functools

import jax
import jax.numpy as jnp
from jax import lax
from jax.experimental import pallas as pl
from jax.experimental.pallas import tpu as pltpu
from jax.experimental.pallas import tpu_sc as plsc

VOCAB = 457
EMB = 64
BATCH = 1024
SEQ = 457
VP = 464

_info = plsc.get_sparse_core_info()
_NC = _info.num_cores
_NS = _info.num_subcores
NW = _NC * _NS

MAXS = SEQ // NW + 1

_mesh = plsc.VectorSubcoreMesh(core_axis_name="c", subcore_axis_name="s")


@functools.partial(
    pl.kernel,
    mesh=_mesh,
    out_type=(
        jax.ShapeDtypeStruct((SEQ, 8, 8, 8, 128), jnp.float32),
        jax.ShapeDtypeStruct((SEQ, 8, 8, 8, 128), jnp.float32),
    ),
    scratch_types=[
        pltpu.VMEM((EMB * VP,), jnp.float32),
        pltpu.VMEM((MAXS * BATCH,), jnp.int32),
        pltpu.VMEM((4, 8, 8, 128), jnp.float32),
        pltpu.VMEM((4, 8, 8, 128), jnp.float32),
        pltpu.SemaphoreType.DMA,
        pltpu.SemaphoreType.DMA,
    ],
    compiler_params=pltpu.CompilerParams(
        use_tc_tiling_on_sc=False, needs_layout_passes=False),
)
def _lookup(seq_f, exp_f, w_flat, o1, o2, table_v, ir_all, b0, b1, ss0, ss1):
    wid = lax.axis_index("s") * _NC + lax.axis_index("c")
    s0 = (wid * SEQ) >> 5
    n = (((wid + 1) * SEQ) >> 5) - s0

    pltpu.sync_copy(w_flat, table_v)

    bufs = (b0, b1)
    ssems = (ss0, ss1)

    tviews = [table_v.at[pl.ds(e * VP, VP)] for e in range(EMB)]

    def compute_half(ioff, h, buf):
        @plsc.parallel_loop(0, 64, unroll=4)
        def gbody(g):
            base = ir_all[pl.ds(ioff + g * 16, 16)]
            bt = g >> 3
            boff = (g & 7) * 16
            for et4 in range(4):
                for e8 in range(8):
                    v = plsc.load_gather(
                        tviews[(h * 4 + et4) * 8 + e8], [base])
                    buf[et4, bt, e8, pl.ds(boff, 16)] = v

    def wait_sc(b):
        pltpu.make_async_copy(
            bufs[b], o1.at[0, pl.ds(0, 4)], ssems[b]).wait()

    def do_s(i, out_hbm, first_s=False):
        for h in range(2):
            if not first_s:
                wait_sc(h)
            compute_half(i * BATCH, h, bufs[h])
            pltpu.async_copy(
                bufs[h], out_hbm.at[s0 + i, pl.ds(h * 4, 4)], ssems[h])

    def run(idx_f, out_hbm, first):
        pltpu.sync_copy(
            idx_f.at[pl.ds(s0 * BATCH, MAXS * BATCH)], ir_all)
        lo = 0
        if first:
            do_s(0, out_hbm, first_s=True)
            lo = 1

        def body(i, c):
            do_s(i, out_hbm)
            return c

        lax.fori_loop(lo, n, body, 0)

    run(seq_f, o1, True)
    run(exp_f, o2, False)
    wait_sc(0)
    wait_sc(1)


def kernel(seqs, exps, W):
    w_pad = jnp.pad(W.T, ((0, 0), (0, VP - VOCAB))).reshape(-1)
    p1, p2 = _lookup(
        seqs.T.reshape(-1), exps.T.reshape(-1), w_pad)

    def unpack(p):
        return p.transpose(2, 4, 0, 1, 3).reshape(BATCH, SEQ, EMB)

    return unpack(p1), unpack(p2)


# speedup vs baseline: 1.0146x; 1.0146x over previous
import functools

import jax
import jax.numpy as jnp
from jax import lax
from jax.experimental import pallas as pl
from jax.experimental.pallas import tpu as pltpu
from jax.experimental.pallas import tpu_sc as plsc

VOCAB = 457
EMB = 64
BATCH = 1024
SEQ = 457
VP = 464

_info = plsc.get_sparse_core_info()
_NC = _info.num_cores
_NS = _info.num_subcores
NW = _NC * _NS

MAXS = SEQ // NW + 1

_mesh = plsc.VectorSubcoreMesh(core_axis_name="c", subcore_axis_name="s")


@functools.partial(
    pl.kernel,
    mesh=_mesh,
    out_type=(
        jax.ShapeDtypeStruct((SEQ, 8, 8, 8, 128), jnp.float32),
        jax.ShapeDtypeStruct((SEQ, 8, 8, 8, 128), jnp.float32),
    ),
    scratch_types=[
        pltpu.VMEM((EMB * VP,), jnp.float32),
        pltpu.VMEM((MAXS * BATCH,), jnp.int32),
        pltpu.VMEM((MAXS * BATCH,), jnp.int32),
        pltpu.VMEM((4, 8, 8, 128), jnp.float32),
        pltpu.VMEM((4, 8, 8, 128), jnp.float32),
        pltpu.SemaphoreType.DMA,
        pltpu.SemaphoreType.DMA,
        pltpu.SemaphoreType.DMA,
        pltpu.SemaphoreType.DMA,
    ],
    compiler_params=pltpu.CompilerParams(
        use_tc_tiling_on_sc=False, needs_layout_passes=False),
)
def _lookup(seq_f, exp_f, w_flat, o1, o2,
            table_v, ir1, ir2, b0, b1, ss0, ss1, is1, is2):
    wid = lax.axis_index("s") * _NC + lax.axis_index("c")
    s0 = (wid * SEQ) >> 5
    n = (((wid + 1) * SEQ) >> 5) - s0

    icp1 = pltpu.make_async_copy(
        seq_f.at[pl.ds(s0 * BATCH, MAXS * BATCH)], ir1, is1)
    icp1.start()
    icp2 = pltpu.make_async_copy(
        exp_f.at[pl.ds(s0 * BATCH, MAXS * BATCH)], ir2, is2)
    icp2.start()
    pltpu.sync_copy(w_flat, table_v)

    bufs = (b0, b1)
    ssems = (ss0, ss1)

    tviews = [table_v.at[pl.ds(e * VP, VP)] for e in range(EMB)]

    def compute_half(ir, ioff, h, buf):
        @plsc.parallel_loop(0, 64, unroll=2)
        def gbody(g):
            base = ir[pl.ds(ioff + g * 16, 16)]
            bt = g >> 3
            boff = (g & 7) * 16
            for et4 in range(4):
                for e8 in range(8):
                    v = plsc.load_gather(
                        tviews[(h * 4 + et4) * 8 + e8], [base])
                    buf[et4, bt, e8, pl.ds(boff, 16)] = v

    def wait_sc(b):
        pltpu.make_async_copy(
            bufs[b], o1.at[0, pl.ds(0, 4)], ssems[b]).wait()

    def do_s(ir, i, out_hbm, first_s=False):
        for h in range(2):
            if not first_s:
                wait_sc(h)
            compute_half(ir, i * BATCH, h, bufs[h])
            pltpu.async_copy(
                bufs[h], out_hbm.at[s0 + i, pl.ds(h * 4, 4)], ssems[h])

    def run(ir, icp, out_hbm, first):
        icp.wait()
        lo = 0
        if first:
            do_s(ir, 0, out_hbm, first_s=True)
            lo = 1

        def body(i, c):
            do_s(ir, i, out_hbm)
            return c

        lax.fori_loop(lo, n, body, 0)

    run(ir1, icp1, o1, True)
    run(ir2, icp2, o2, False)
    wait_sc(0)
    wait_sc(1)


def kernel(seqs, exps, W):
    w_pad = jnp.pad(W.T, ((0, 0), (0, VP - VOCAB))).reshape(-1)
    p1, p2 = _lookup(
        seqs.T.reshape(-1), exps.T.reshape(-1), w_pad)

    def unpack(p):
        return p.transpose(2, 4, 0, 1, 3).reshape(BATCH, SEQ, EMB)

    return unpack(p1), unpack(p2)
